# baseline (speedup 1.0000x reference)
import jax
import jax.numpy as jnp
from jax.experimental import pallas as pl
from jax.experimental.pallas import tpu as pltpu

_B, _C, _H, _W = 8, 3, 128, 128
_K = 100
_HW = _H * _W
_NR = _C * _H
_KPAD = 104
_DOWN_RATIO = 4.0
_IMG_W, _IMG_H = 512.0, 512.0
_DET_THRESHOLD = 0.2
_DIM_MEAN = jnp.array(
    [[3.88, 1.53, 1.63], [0.84, 1.76, 0.66], [1.76, 1.74, 0.60]],
    dtype=jnp.float32,
)


def _pp_kernel(heat_ref, reg_ref, out_ref, s_ref, rm_ref):
    x = heat_ref[...].reshape(_B * _C, _H, _W)
    ninf = jnp.float32(-jnp.inf)
    pad_row = jnp.full((_B * _C, 1, _W), ninf, jnp.float32)
    up = jnp.concatenate([x[:, 1:, :], pad_row], axis=1)
    dn = jnp.concatenate([pad_row, x[:, :-1, :]], axis=1)
    v = jnp.maximum(jnp.maximum(x, up), dn)
    pad_col = jnp.full((_B * _C, _H, 1), ninf, jnp.float32)
    lf = jnp.concatenate([v[:, :, 1:], pad_col], axis=2)
    rt = jnp.concatenate([pad_col, v[:, :, :-1]], axis=2)
    hmax = jnp.maximum(jnp.maximum(v, lf), rt)
    s = jnp.where(hmax == x, x, 0.0).reshape(_B, _NR, _W)
    s_ref[...] = s
    rm_ref[...] = jnp.max(s.reshape(_B, _C, _H, _W), axis=3)

    ii2 = (
        jax.lax.broadcasted_iota(jnp.int32, (_C, _H), 0) * _H
        + jax.lax.broadcasted_iota(jnp.int32, (_C, _H), 1)
    )
    li = jax.lax.broadcasted_iota(jnp.int32, (1, _W), 1)
    big = jnp.int32(1 << 30)

    def body(k, carry):
        for b in range(_B):
            rmb = rm_ref[b]
            m = jnp.max(rmb)
            r = jnp.min(jnp.where(rmb == m, ii2, big))
            row = s_ref[b, pl.ds(r, 1), :]
            c = jnp.min(jnp.where(row == m, li, big))
            flat = r * _W + c
            y = jax.lax.rem(r, _H)
            oh = li == c
            outrow = jnp.where(li == 13, m, jnp.float32(0.0))
            outrow = jnp.where(li == 14, flat.astype(jnp.float32), outrow)
            for ch in range(13):
                rrow = reg_ref[b, ch, pl.ds(y, 1), :]
                vv = jnp.sum(jnp.where(oh, rrow, 0.0))
                outrow = jnp.where(li == ch, vv, outrow)
            out_ref[b, pl.ds(k, 1), :] = outrow
            nrow = jnp.where(oh, jnp.float32(-1.0), row)
            s_ref[b, pl.ds(r, 1), :] = nrow
            nm = jnp.max(nrow)
            rhi = jax.lax.div(r, _H)
            rmrow = rm_ref[b, pl.ds(rhi, 1), :]
            rm_ref[b, pl.ds(rhi, 1), :] = jnp.where(li == y, nm, rmrow)
        return carry

    jax.lax.fori_loop(0, _K, body, jnp.int32(0))


def kernel(pred_heatmap, pred_regression, calib_P):
    reg13 = jnp.concatenate(
        [pred_regression[:, 0:11], pred_regression[:, 27:29]], axis=1
    )
    raw = pl.pallas_call(
        _pp_kernel,
        out_shape=jax.ShapeDtypeStruct((_B, _KPAD, _W), jnp.float32),
        scratch_shapes=[
            pltpu.VMEM((_B, _NR, _W), jnp.float32),
            pltpu.VMEM((_B, _C, _H), jnp.float32),
        ],
    )(pred_heatmap, reg13)

    t = raw[:, :_K, :15].reshape(_B * _K, 15)
    pois = t[:, :13]
    scores = t[:, 13]
    idx = t[:, 14].astype(jnp.int32)
    clses_f = (idx // _HW).astype(jnp.float32)
    sp = idx % _HW
    ys = (sp // _W).astype(jnp.float32)
    xs = (sp % _W).astype(jnp.float32)
    points = jnp.stack([xs, ys], axis=1) + 0.5

    reg2d = jax.nn.relu(pois[:, 0:4])
    x1 = jnp.clip((points[:, 0] - reg2d[:, 0]) * _DOWN_RATIO, 0.0, _IMG_W - 1.0)
    y1 = jnp.clip((points[:, 1] - reg2d[:, 1]) * _DOWN_RATIO, 0.0, _IMG_H - 1.0)
    x2 = jnp.clip((points[:, 0] + reg2d[:, 2]) * _DOWN_RATIO, 0.0, _IMG_W - 1.0)
    y2 = jnp.clip((points[:, 1] + reg2d[:, 3]) * _DOWN_RATIO, 0.0, _IMG_H - 1.0)
    box2d = jnp.stack([x1, y1, x2, y2], axis=1)
    offset3d = pois[:, 4:6]
    dims = jnp.exp(jnp.clip(pois[:, 6:9], -5.0, 5.0)) * _DIM_MEAN[
        idx // _HW
    ]
    depth = 1.0 / jax.nn.sigmoid(pois[:, 9]) - 1.0
    unc = jnp.exp(jnp.clip(pois[:, 10], -10.0, 10.0))
    proj = (points + offset3d) * _DOWN_RATIO
    batch_idxs = jnp.repeat(jnp.arange(_B), _K)
    P = calib_P[batch_idxs]
    fu = P[:, 0, 0]
    cu = P[:, 0, 2]
    fv = P[:, 1, 1]
    cv = P[:, 1, 2]
    x3d = (proj[:, 0] - cu) * depth / fu
    y3d = (proj[:, 1] - cv) * depth / fv
    alpha = jnp.arctan2(pois[:, 11], pois[:, 12])
    roty = alpha + jnp.arctan2(x3d, depth)
    conf = 1.0 / (1.0 + unc)
    final_scores = scores * conf
    final_scores = jnp.where(final_scores >= _DET_THRESHOLD, final_scores, 0.0)
    return jnp.concatenate(
        [
            clses_f[:, None],
            alpha[:, None],
            box2d,
            dims,
            jnp.stack([x3d, y3d, depth], axis=1),
            roty[:, None],
            final_scores[:, None],
        ],
        axis=1,
    )


# speedup vs baseline: 1.1574x; 1.1574x over previous
import jax
import jax.numpy as jnp
from jax.experimental import pallas as pl
from jax.experimental.pallas import tpu as pltpu

_B, _C, _H, _W = 8, 3, 128, 128
_K = 100
_HW = _H * _W
_NR = _C * _H
_KPAD = 104
_DOWN_RATIO = 4.0
_IMG_W, _IMG_H = 512.0, 512.0
_DET_THRESHOLD = 0.2
_DIM_MEAN = jnp.array(
    [[3.88, 1.53, 1.63], [0.84, 1.76, 0.66], [1.76, 1.74, 0.60]],
    dtype=jnp.float32,
)


def _pp_kernel(heat_ref, reg_ref, *refs):
    out_refs = refs[:_B]
    s_refs = refs[_B : 2 * _B]
    rm_refs = refs[2 * _B :]

    x = heat_ref[...].reshape(_B * _C, _H, _W)
    ninf = jnp.float32(-jnp.inf)
    pad_row = jnp.full((_B * _C, 1, _W), ninf, jnp.float32)
    up = jnp.concatenate([x[:, 1:, :], pad_row], axis=1)
    dn = jnp.concatenate([pad_row, x[:, :-1, :]], axis=1)
    v = jnp.maximum(jnp.maximum(x, up), dn)
    pad_col = jnp.full((_B * _C, _H, 1), ninf, jnp.float32)
    lf = jnp.concatenate([v[:, :, 1:], pad_col], axis=2)
    rt = jnp.concatenate([pad_col, v[:, :, :-1]], axis=2)
    hmax = jnp.maximum(jnp.maximum(v, lf), rt)
    s = jnp.where(hmax == x, x, 0.0).reshape(_B, _NR, _W)
    for b in range(_B):
        s_refs[b][...] = s[b]
        rm_refs[b][...] = jnp.max(s[b].reshape(_C, _H, _W), axis=2)

    ii2 = (
        jax.lax.broadcasted_iota(jnp.int32, (_C, _H), 0) * _H
        + jax.lax.broadcasted_iota(jnp.int32, (_C, _H), 1)
    )
    li = jax.lax.broadcasted_iota(jnp.int32, (1, _W), 1)
    big = jnp.int32(1 << 30)

    def body(k, carry):
        for b in range(_B):
            rmb = rm_refs[b][...]
            m = jnp.max(rmb)
            r = jnp.min(jnp.where(rmb == m, ii2, big))
            row = s_refs[b][pl.ds(r, 1), :]
            c = jnp.min(jnp.where(row == m, li, big))
            flat = r * _W + c
            y = jax.lax.rem(r, _H)
            oh = li == c
            outrow = jnp.where(li == 13, m, jnp.float32(0.0))
            outrow = jnp.where(li == 14, flat.astype(jnp.float32), outrow)
            for ch in range(13):
                rrow = reg_ref[b, ch, pl.ds(y, 1), :]
                vv = jnp.sum(jnp.where(oh, rrow, 0.0))
                outrow = jnp.where(li == ch, vv, outrow)
            out_refs[b][pl.ds(k, 1), :] = outrow
            nrow = jnp.where(oh, jnp.float32(-1.0), row)
            s_refs[b][pl.ds(r, 1), :] = nrow
            nm = jnp.max(nrow)
            rhi = jax.lax.div(r, _H)
            rmrow = rm_refs[b][pl.ds(rhi, 1), :]
            rm_refs[b][pl.ds(rhi, 1), :] = jnp.where(li == y, nm, rmrow)
        return carry

    jax.lax.fori_loop(0, _K, body, jnp.int32(0))


def kernel(pred_heatmap, pred_regression, calib_P):
    reg13 = jnp.concatenate(
        [pred_regression[:, 0:11], pred_regression[:, 27:29]], axis=1
    )
    raws = pl.pallas_call(
        _pp_kernel,
        out_shape=[
            jax.ShapeDtypeStruct((_KPAD, _W), jnp.float32) for _ in range(_B)
        ],
        scratch_shapes=(
            [pltpu.VMEM((_NR, _W), jnp.float32) for _ in range(_B)]
            + [pltpu.VMEM((_C, _H), jnp.float32) for _ in range(_B)]
        ),
    )(pred_heatmap, reg13)
    raw = jnp.stack(raws, axis=0)

    t = raw[:, :_K, :15].reshape(_B * _K, 15)
    pois = t[:, :13]
    scores = t[:, 13]
    idx = t[:, 14].astype(jnp.int32)
    clses_f = (idx // _HW).astype(jnp.float32)
    sp = idx % _HW
    ys = (sp // _W).astype(jnp.float32)
    xs = (sp % _W).astype(jnp.float32)
    points = jnp.stack([xs, ys], axis=1) + 0.5

    reg2d = jax.nn.relu(pois[:, 0:4])
    x1 = jnp.clip((points[:, 0] - reg2d[:, 0]) * _DOWN_RATIO, 0.0, _IMG_W - 1.0)
    y1 = jnp.clip((points[:, 1] - reg2d[:, 1]) * _DOWN_RATIO, 0.0, _IMG_H - 1.0)
    x2 = jnp.clip((points[:, 0] + reg2d[:, 2]) * _DOWN_RATIO, 0.0, _IMG_W - 1.0)
    y2 = jnp.clip((points[:, 1] + reg2d[:, 3]) * _DOWN_RATIO, 0.0, _IMG_H - 1.0)
    box2d = jnp.stack([x1, y1, x2, y2], axis=1)
    offset3d = pois[:, 4:6]
    dims = jnp.exp(jnp.clip(pois[:, 6:9], -5.0, 5.0)) * _DIM_MEAN[
        idx // _HW
    ]
    depth = 1.0 / jax.nn.sigmoid(pois[:, 9]) - 1.0
    unc = jnp.exp(jnp.clip(pois[:, 10], -10.0, 10.0))
    proj = (points + offset3d) * _DOWN_RATIO
    batch_idxs = jnp.repeat(jnp.arange(_B), _K)
    P = calib_P[batch_idxs]
    fu = P[:, 0, 0]
    cu = P[:, 0, 2]
    fv = P[:, 1, 1]
    cv = P[:, 1, 2]
    x3d = (proj[:, 0] - cu) * depth / fu
    y3d = (proj[:, 1] - cv) * depth / fv
    alpha = jnp.arctan2(pois[:, 11], pois[:, 12])
    roty = alpha + jnp.arctan2(x3d, depth)
    conf = 1.0 / (1.0 + unc)
    final_scores = scores * conf
    final_scores = jnp.where(final_scores >= _DET_THRESHOLD, final_scores, 0.0)
    return jnp.concatenate(
        [
            clses_f[:, None],
            alpha[:, None],
            box2d,
            dims,
            jnp.stack([x3d, y3d, depth], axis=1),
            roty[:, None],
            final_scores[:, None],
        ],
        axis=1,
    )
